# baseline (speedup 1.0000x reference)
import functools

import jax
import jax.numpy as jnp
from jax import lax
from jax.experimental import pallas as pl
from jax.experimental.pallas import tpu as pltpu
from jax.experimental.pallas import tpu_sc as plsc

N_CLS = 1000
N_CTX = 16
D = 768
SEQ = 77
SUF = SEQ - 1 - N_CTX


def _sc_concat(init, token_prefix, token_suffix):
    info = plsc.get_sparse_core_info()
    NC, NS = info.num_cores, info.num_subcores
    NW = NC * NS

    mesh = plsc.VectorSubcoreMesh(core_axis_name="c", subcore_axis_name="s")

    @functools.partial(
        pl.kernel,
        mesh=mesh,
        out_type=jax.ShapeDtypeStruct((N_CLS, SEQ, D), jnp.float32),
        scratch_types=[
            pltpu.VMEM((N_CTX, D), jnp.float32),
            pltpu.VMEM((1, D), jnp.float32),
            pltpu.VMEM((SUF, D), jnp.float32),
        ],
        compiler_params=pltpu.CompilerParams(use_tc_tiling_on_sc=False),
    )
    def k(ctx_hbm, pre_hbm, suf_hbm, out_hbm, ctx_v, pre_v, suf_v):
        wid = lax.axis_index("s") * NC + lax.axis_index("c")
        lo = (wid * N_CLS) // NW
        hi = ((wid + 1) * N_CLS) // NW

        pltpu.sync_copy(ctx_hbm, ctx_v)

        def body(c, carry):
            pltpu.sync_copy(pre_hbm.at[c], pre_v)
            pltpu.sync_copy(pre_v, out_hbm.at[c, pl.ds(0, 1)])
            pltpu.sync_copy(ctx_v, out_hbm.at[c, pl.ds(1, N_CTX)])
            pltpu.sync_copy(suf_hbm.at[c], suf_v)
            pltpu.sync_copy(suf_v, out_hbm.at[c, pl.ds(1 + N_CTX, SUF)])
            return carry

        lax.fori_loop(lo, hi, body, 0)

    return k(init, token_prefix, token_suffix)


def kernel(init, token_prefix, token_suffix):
    return _sc_concat(init, token_prefix, token_suffix)
